# baseline (speedup 1.0000x reference)
import jax
import jax.numpy as jnp
from jax import lax
from jax.experimental import pallas as pl
from jax.experimental.pallas import tpu as pltpu
from jax.experimental.pallas import tpu_sc as plsc

B, L, V, D = 4096, 200, 1000000, 64
N = B * L
NW = 32
PER_W = N // NW
IDXW = 128
CHUNK_IR = 2
CHUNK = CHUNK_IR * IDXW
NCHUNK = PER_W // CHUNK
IR_PER_W = PER_W // IDXW
SCALE = float(D) ** 0.5
LANES = 16
GPC = CHUNK // LANES
TCOL = N // IDXW


def _sc_body(table, seq2d, pos2d, neg2d,
             seq_out, pos_out, neg_out, ist_out,
             idx_all, rows_v0, rows_v1, tr_v0, tr_v1, ist_v0, ist_v1,
             gsem0, gsem1, osem0, osem1):
    wid = lax.axis_index("s") * 2 + lax.axis_index("c")
    rows_vs = (rows_v0, rows_v1)
    tr_vs = (tr_v0, tr_v1)
    ist_vs = (ist_v0, ist_v1)
    gsems = (gsem0, gsem1)
    osems = (osem0, osem1)
    iota = lax.iota(jnp.int32, LANES)

    def run_array(idx2d, out_hbm, scaled, want_ist):
        pltpu.sync_copy(idx2d.at[pl.ds(wid * IR_PER_W, IR_PER_W)], idx_all)

        def fetch(g, b):
            for j in range(CHUNK_IR):
                pltpu.async_copy(
                    table.at[idx_all.at[g * CHUNK_IR + j]],
                    rows_vs[b].at[pl.ds(j * IDXW, IDXW)],
                    gsems[b],
                )

        def wait_gather(b):
            pltpu.make_async_copy(
                table.at[pl.ds(0, CHUNK)], rows_vs[b], gsems[b]).wait()

        def fire_out(g, b):
            tc0 = wid * IR_PER_W + g * CHUNK_IR
            pltpu.async_copy(
                tr_vs[b], out_hbm.at[:, pl.ds(tc0, CHUNK_IR), :, :],
                osems[b])
            if want_ist:
                base = wid * PER_W + g * CHUNK
                pltpu.async_copy(ist_vs[b], ist_out.at[pl.ds(base, CHUNK)],
                                 osems[b])

        def wait_out(b):
            pltpu.make_async_copy(
                tr_vs[b], out_hbm.at[:, pl.ds(0, CHUNK_IR), :, :],
                osems[b]).wait()
            if want_ist:
                pltpu.make_async_copy(
                    ist_vs[b], ist_out.at[pl.ds(0, CHUNK)], osems[b]).wait()

        def process(g, b):
            def m_body(m, c2):
                irow = g * CHUNK_IR + m // (IDXW // LANES)
                ioff = (m % (IDXW // LANES)) * LANES
                v = idx_all[irow, pl.ds(ioff, LANES)]
                mult = jnp.where(
                    v != 0,
                    jnp.float32(SCALE if scaled else 1.0),
                    jnp.float32(0.0))
                if want_ist:
                    ist_vs[b][pl.ds(m * LANES, LANES)] = jnp.where(
                        v != 0, jnp.float32(1.0), jnp.float32(0.0))
                r_vec = iota + m * LANES
                tci = m // (IDXW // LANES)
                rc0 = (m % (IDXW // LANES)) * LANES
                for j in range(D):
                    j_vec = jnp.full((LANES,), j, jnp.int32)
                    x = plsc.load_gather(rows_vs[b], [r_vec, j_vec])
                    tr_vs[b][j // 8, tci, j % 8, pl.ds(rc0, LANES)] = (
                        x * mult)
                return c2
            lax.fori_loop(0, GPC, m_body, 0)

        fetch(0, 0)

        def pair_body(g2, carry):
            for b in range(2):
                g = g2 * 2 + b
                wait_gather(b)

                @pl.when(g + 1 < NCHUNK)
                def _prefetch():
                    fetch(g + 1, 1 - b)

                @pl.when(g >= 2)
                def _reuse_wait():
                    wait_out(b)

                process(g, b)
                fire_out(g, b)
            return carry

        lax.fori_loop(0, NCHUNK // 2, pair_body, 0)
        wait_out(0)
        wait_out(1)

    run_array(seq2d, seq_out, True, False)
    run_array(pos2d, pos_out, False, True)
    run_array(neg2d, neg_out, False, False)


@jax.jit
def _sc_call(table, seq2d, pos2d, neg2d):
    mesh = plsc.VectorSubcoreMesh(core_axis_name="c", subcore_axis_name="s")
    q4d = jax.ShapeDtypeStruct((D // 8, TCOL, 8, IDXW), jnp.float32)
    f = pl.kernel(
        _sc_body,
        out_type=(
            q4d,
            q4d,
            q4d,
            jax.ShapeDtypeStruct((N,), jnp.float32),
        ),
        mesh=mesh,
        scratch_types=[
            pltpu.VMEM((IR_PER_W, IDXW), jnp.int32),
            pltpu.VMEM((CHUNK, D), jnp.float32),
            pltpu.VMEM((CHUNK, D), jnp.float32),
            pltpu.VMEM((D // 8, CHUNK_IR, 8, IDXW), jnp.float32),
            pltpu.VMEM((D // 8, CHUNK_IR, 8, IDXW), jnp.float32),
            pltpu.VMEM((CHUNK,), jnp.float32),
            pltpu.VMEM((CHUNK,), jnp.float32),
            pltpu.SemaphoreType.DMA,
            pltpu.SemaphoreType.DMA,
            pltpu.SemaphoreType.DMA,
            pltpu.SemaphoreType.DMA,
        ],
        compiler_params=pltpu.CompilerParams(
            use_tc_tiling_on_sc=False, needs_layout_passes=False),
    )
    return f(table, seq2d, pos2d, neg2d)


def kernel(seq_ids, pos_ids, neg_ids, item_embedding_table):
    seq2d = seq_ids.reshape(N // IDXW, IDXW)
    pos2d = pos_ids.reshape(N // IDXW, IDXW)
    neg2d = neg_ids.reshape(N // IDXW, IDXW)
    seq4, pos4, neg4, istarget = _sc_call(
        item_embedding_table, seq2d, pos2d, neg2d)

    def untile(q):
        return q.transpose(1, 3, 0, 2).reshape(N, D)

    return untile(seq4), untile(pos4), untile(neg4), istarget


# speedup vs baseline: 1.7678x; 1.7678x over previous
import jax
import jax.numpy as jnp
from jax import lax
from jax.experimental import pallas as pl
from jax.experimental.pallas import tpu as pltpu
from jax.experimental.pallas import tpu_sc as plsc

B, L, V, D = 4096, 200, 1000000, 64
N = B * L
NW = 32
PER_W = N // NW
IDXW = 128
CHUNK_IR = 2
CHUNK = CHUNK_IR * IDXW
NCHUNK = PER_W // CHUNK
IR_PER_W = PER_W // IDXW
SCALE = float(D) ** 0.5
LANES = 16
GPC = CHUNK // LANES
TCOL = N // IDXW


def _sc_body(table, seq2d, pos2d, neg2d,
             seq_out, pos_out, neg_out, ist_out,
             idx_all, rows_v0, rows_v1, tr_v0, tr_v1, ist_v0, ist_v1,
             gsem0, gsem1, osem0, osem1):
    wid = lax.axis_index("s") * 2 + lax.axis_index("c")
    rows_vs = (rows_v0, rows_v1)
    tr_vs = (tr_v0, tr_v1)
    ist_vs = (ist_v0, ist_v1)
    gsems = (gsem0, gsem1)
    osems = (osem0, osem1)
    iota = lax.iota(jnp.int32, LANES)

    def run_array(idx2d, out_hbm, scaled, want_ist):
        pltpu.sync_copy(idx2d.at[pl.ds(wid * IR_PER_W, IR_PER_W)], idx_all)

        def fetch(g, b):
            for j in range(CHUNK_IR):
                pltpu.async_copy(
                    table.at[idx_all.at[g * CHUNK_IR + j]],
                    rows_vs[b].at[pl.ds(j * IDXW, IDXW)],
                    gsems[b],
                )

        def wait_gather(b):
            pltpu.make_async_copy(
                table.at[pl.ds(0, CHUNK)], rows_vs[b], gsems[b]).wait()

        def fire_out(g, b):
            tc0 = wid * IR_PER_W + g * CHUNK_IR
            pltpu.async_copy(
                tr_vs[b], out_hbm.at[:, pl.ds(tc0, CHUNK_IR), :, :],
                osems[b])
            if want_ist:
                base = wid * PER_W + g * CHUNK
                pltpu.async_copy(ist_vs[b], ist_out.at[pl.ds(base, CHUNK)],
                                 osems[b])

        def wait_out(b):
            pltpu.make_async_copy(
                tr_vs[b], out_hbm.at[:, pl.ds(0, CHUNK_IR), :, :],
                osems[b]).wait()
            if want_ist:
                pltpu.make_async_copy(
                    ist_vs[b], ist_out.at[pl.ds(0, CHUNK)], osems[b]).wait()

        def process(g, b):
            def m_body(m, c2):
                irow = g * CHUNK_IR + m // (IDXW // LANES)
                ioff = (m % (IDXW // LANES)) * LANES
                v = idx_all[irow, pl.ds(ioff, LANES)]
                mult = jnp.where(
                    v != 0,
                    jnp.float32(SCALE if scaled else 1.0),
                    jnp.float32(0.0))
                if want_ist:
                    ist_vs[b][pl.ds(m * LANES, LANES)] = jnp.where(
                        v != 0, jnp.float32(1.0), jnp.float32(0.0))
                r_vec = iota + m * LANES
                tci = m // (IDXW // LANES)
                rc0 = (m % (IDXW // LANES)) * LANES

                @plsc.parallel_loop(0, D, unroll=8)
                def _j_loop(j):
                    j_vec = jnp.full((LANES,), 0, jnp.int32) + j
                    x = plsc.load_gather(rows_vs[b], [r_vec, j_vec])
                    tr_vs[b][j // 8, tci, j % 8, pl.ds(rc0, LANES)] = (
                        x * mult)
                return c2
            lax.fori_loop(0, GPC, m_body, 0)

        fetch(0, 0)

        def pair_body(g2, carry):
            for b in range(2):
                g = g2 * 2 + b
                wait_gather(b)

                @pl.when(g + 1 < NCHUNK)
                def _prefetch():
                    fetch(g + 1, 1 - b)

                @pl.when(g >= 2)
                def _reuse_wait():
                    wait_out(b)

                process(g, b)
                fire_out(g, b)
            return carry

        lax.fori_loop(0, NCHUNK // 2, pair_body, 0)
        wait_out(0)
        wait_out(1)

    run_array(seq2d, seq_out, True, False)
    run_array(pos2d, pos_out, False, True)
    run_array(neg2d, neg_out, False, False)


@jax.jit
def _sc_call(table, seq2d, pos2d, neg2d):
    mesh = plsc.VectorSubcoreMesh(core_axis_name="c", subcore_axis_name="s")
    q4d = jax.ShapeDtypeStruct((D // 8, TCOL, 8, IDXW), jnp.float32)
    f = pl.kernel(
        _sc_body,
        out_type=(
            q4d,
            q4d,
            q4d,
            jax.ShapeDtypeStruct((N,), jnp.float32),
        ),
        mesh=mesh,
        scratch_types=[
            pltpu.VMEM((IR_PER_W, IDXW), jnp.int32),
            pltpu.VMEM((CHUNK, D), jnp.float32),
            pltpu.VMEM((CHUNK, D), jnp.float32),
            pltpu.VMEM((D // 8, CHUNK_IR, 8, IDXW), jnp.float32),
            pltpu.VMEM((D // 8, CHUNK_IR, 8, IDXW), jnp.float32),
            pltpu.VMEM((CHUNK,), jnp.float32),
            pltpu.VMEM((CHUNK,), jnp.float32),
            pltpu.SemaphoreType.DMA,
            pltpu.SemaphoreType.DMA,
            pltpu.SemaphoreType.DMA,
            pltpu.SemaphoreType.DMA,
        ],
        compiler_params=pltpu.CompilerParams(
            use_tc_tiling_on_sc=False, needs_layout_passes=False,
            disable_bounds_checks=True),
    )
    return f(table, seq2d, pos2d, neg2d)


def kernel(seq_ids, pos_ids, neg_ids, item_embedding_table):
    seq2d = seq_ids.reshape(N // IDXW, IDXW)
    pos2d = pos_ids.reshape(N // IDXW, IDXW)
    neg2d = neg_ids.reshape(N // IDXW, IDXW)
    seq4, pos4, neg4, istarget = _sc_call(
        item_embedding_table, seq2d, pos2d, neg2d)

    def untile(q):
        return q.transpose(1, 3, 0, 2).reshape(N, D)

    return untile(seq4), untile(pos4), untile(neg4), istarget


# speedup vs baseline: 4.0980x; 2.3182x over previous
import jax
import jax.numpy as jnp
from jax import lax
from jax.experimental import pallas as pl
from jax.experimental.pallas import tpu as pltpu
from jax.experimental.pallas import tpu_sc as plsc

B, L, V, D = 4096, 200, 1000000, 64
N = B * L
NW = 32
PER_W = N // NW
IDXW = 128
CHUNK_IR = 2
CHUNK = CHUNK_IR * IDXW
NCHUNK = PER_W // CHUNK
IR_PER_W = PER_W // IDXW
SCALE = float(D) ** 0.5
LANES = 16
GPC = CHUNK // LANES
GPR = IDXW // LANES
TCOL = N // IDXW


def _sc_body(table, seq2d, pos2d, neg2d,
             seq_out, pos_out, neg_out, ist_out,
             idx_all, rows_v0, rows_v1, tr_v0, tr_v1, swz_v,
             ist_v0, ist_v1, gsem0, gsem1, osem0, osem1):
    wid = lax.axis_index("s") * 2 + lax.axis_index("c")
    rows_vs = (rows_v0, rows_v1)
    tr_vs = (tr_v0, tr_v1)
    ist_vs = (ist_v0, ist_v1)
    gsems = (gsem0, gsem1)
    osems = (osem0, osem1)
    iota = lax.iota(jnp.int32, LANES)

    def run_array(idx2d, out_hbm, scaled, want_ist):
        pltpu.sync_copy(idx2d.at[pl.ds(wid * IR_PER_W, IR_PER_W)], idx_all)

        def fetch(g, b):
            for j in range(CHUNK_IR):
                pltpu.async_copy(
                    table.at[idx_all.at[g * CHUNK_IR + j]],
                    rows_vs[b].at[pl.ds(j * IDXW, IDXW)],
                    gsems[b],
                )

        def wait_gather(b):
            pltpu.make_async_copy(
                table.at[pl.ds(0, CHUNK)], rows_vs[b], gsems[b]).wait()

        def fire_out(g, b):
            tc0 = wid * IR_PER_W + g * CHUNK_IR
            pltpu.async_copy(
                tr_vs[b], out_hbm.at[:, pl.ds(tc0, CHUNK_IR), :, :],
                osems[b])
            if want_ist:
                base = wid * PER_W + g * CHUNK
                pltpu.async_copy(ist_vs[b], ist_out.at[pl.ds(base, CHUNK)],
                                 osems[b])

        def wait_out(b):
            pltpu.make_async_copy(
                tr_vs[b], out_hbm.at[:, pl.ds(0, CHUNK_IR), :, :],
                osems[b]).wait()
            if want_ist:
                pltpu.make_async_copy(
                    ist_vs[b], ist_out.at[pl.ds(0, CHUNK)], osems[b]).wait()

        def process(g, b):
            @plsc.parallel_loop(0, CHUNK, unroll=4)
            def _row_loop(r):
                xv = iota ^ (r % LANES)
                for c in range(D // LANES):
                    x = rows_vs[b][r, pl.ds(c * LANES, LANES)]
                    plsc.store_scatter(
                        swz_v, [xv + (r * D + c * LANES)], x)

            def m_body(m, c2):
                irow = g * CHUNK_IR + m // GPR
                ioff = (m % GPR) * LANES
                v = idx_all[irow, pl.ds(ioff, LANES)]
                mult = jnp.where(
                    v != 0,
                    jnp.float32(SCALE if scaled else 1.0),
                    jnp.float32(0.0))
                if want_ist:
                    ist_vs[b][pl.ds(m * LANES, LANES)] = jnp.where(
                        v != 0, jnp.float32(1.0), jnp.float32(0.0))
                tci = m // GPR
                rc0 = (m % GPR) * LANES
                base_vec = iota * D + (m * LANES * D)

                @plsc.parallel_loop(0, D, unroll=8)
                def _j_loop(j):
                    addr = base_vec + (((j % LANES) ^ iota)
                                       + (j // LANES) * LANES)
                    x = plsc.load_gather(swz_v, [addr])
                    tr_vs[b][j // 8, tci, j % 8, pl.ds(rc0, LANES)] = (
                        x * mult)
                return c2
            lax.fori_loop(0, GPC, m_body, 0)

        fetch(0, 0)

        def pair_body(g2, carry):
            for b in range(2):
                g = g2 * 2 + b
                wait_gather(b)

                @pl.when(g + 1 < NCHUNK)
                def _prefetch():
                    fetch(g + 1, 1 - b)

                @pl.when(g >= 2)
                def _reuse_wait():
                    wait_out(b)

                process(g, b)
                fire_out(g, b)
            return carry

        lax.fori_loop(0, NCHUNK // 2, pair_body, 0)
        wait_out(0)
        wait_out(1)

    run_array(seq2d, seq_out, True, False)
    run_array(pos2d, pos_out, False, True)
    run_array(neg2d, neg_out, False, False)


@jax.jit
def _sc_call(table, seq2d, pos2d, neg2d):
    mesh = plsc.VectorSubcoreMesh(core_axis_name="c", subcore_axis_name="s")
    q4d = jax.ShapeDtypeStruct((D // 8, TCOL, 8, IDXW), jnp.float32)
    f = pl.kernel(
        _sc_body,
        out_type=(
            q4d,
            q4d,
            q4d,
            jax.ShapeDtypeStruct((N,), jnp.float32),
        ),
        mesh=mesh,
        scratch_types=[
            pltpu.VMEM((IR_PER_W, IDXW), jnp.int32),
            pltpu.VMEM((CHUNK, D), jnp.float32),
            pltpu.VMEM((CHUNK, D), jnp.float32),
            pltpu.VMEM((D // 8, CHUNK_IR, 8, IDXW), jnp.float32),
            pltpu.VMEM((D // 8, CHUNK_IR, 8, IDXW), jnp.float32),
            pltpu.VMEM((CHUNK * D,), jnp.float32),
            pltpu.VMEM((CHUNK,), jnp.float32),
            pltpu.VMEM((CHUNK,), jnp.float32),
            pltpu.SemaphoreType.DMA,
            pltpu.SemaphoreType.DMA,
            pltpu.SemaphoreType.DMA,
            pltpu.SemaphoreType.DMA,
        ],
        compiler_params=pltpu.CompilerParams(
            use_tc_tiling_on_sc=False, needs_layout_passes=False,
            disable_bounds_checks=True),
    )
    return f(table, seq2d, pos2d, neg2d)


def kernel(seq_ids, pos_ids, neg_ids, item_embedding_table):
    seq2d = seq_ids.reshape(N // IDXW, IDXW)
    pos2d = pos_ids.reshape(N // IDXW, IDXW)
    neg2d = neg_ids.reshape(N // IDXW, IDXW)
    seq4, pos4, neg4, istarget = _sc_call(
        item_embedding_table, seq2d, pos2d, neg2d)

    def untile(q):
        return q.transpose(1, 3, 0, 2).reshape(N, D)

    return untile(seq4), untile(pos4), untile(neg4), istarget
